# baseline (speedup 1.0000x reference)
import functools
import jax
import jax.numpy as jnp
from jax import lax
from jax.experimental import pallas as pl
from jax.experimental.pallas import tpu as pltpu
from jax.experimental.pallas import tpu_sc as plsc

VOCAB = 100000
EMBED_DIM = 300
BATCH = 4096

NUM_CORES = 2
NUM_SUBCORES = 16
LANES = 16
NUM_WORKERS = NUM_CORES * NUM_SUBCORES
B_PER_W = BATCH // NUM_WORKERS

_mesh = plsc.VectorSubcoreMesh(core_axis_name="c", subcore_axis_name="s")


@functools.partial(
    pl.kernel,
    mesh=_mesh,
    out_type=jax.ShapeDtypeStruct((BATCH, EMBED_DIM), jnp.float32),
    scratch_types=[
        pltpu.VMEM((B_PER_W,), jnp.int32),
        pltpu.SemaphoreType.DMA,
    ],
    compiler_params=pltpu.CompilerParams(use_tc_tiling_on_sc=False),
)
def _gather_kernel(idx_hbm, table_hbm, out_hbm, idx_v, sem):
    wid = lax.axis_index("s") * NUM_CORES + lax.axis_index("c")
    base = wid * B_PER_W
    pltpu.sync_copy(idx_hbm.at[pl.ds(base, B_PER_W)], idx_v)

    def fire(c, carry):
        vec = idx_v[pl.ds(c * LANES, LANES)]
        for j in range(LANES):
            i = vec[j]
            pltpu.make_async_copy(
                table_hbm.at[pl.ds(i, 1)],
                out_hbm.at[pl.ds(base + c * LANES + j, 1)],
                sem,
            ).start()
        return carry

    lax.fori_loop(0, B_PER_W // LANES, fire, 0)
    pltpu.make_async_copy(
        table_hbm.at[pl.ds(0, B_PER_W)],
        out_hbm.at[pl.ds(base, B_PER_W)],
        sem,
    ).wait()


def kernel(news_ids, embedding_table):
    idx = news_ids.astype(jnp.int32)
    return _gather_kernel(idx, embedding_table)


# speedup vs baseline: 2.8707x; 2.8707x over previous
import functools
import jax
import jax.numpy as jnp
from jax import lax
from jax.experimental import pallas as pl
from jax.experimental.pallas import tpu as pltpu
from jax.experimental.pallas import tpu_sc as plsc

VOCAB = 100000
EMBED_DIM = 300
BATCH = 4096

NUM_CORES = 2
NUM_SUBCORES = 16
LANES = 16
NUM_WORKERS = NUM_CORES * NUM_SUBCORES
B_PER_W = BATCH // NUM_WORKERS

_mesh = plsc.VectorSubcoreMesh(core_axis_name="c", subcore_axis_name="s")


@functools.partial(
    pl.kernel,
    mesh=_mesh,
    out_type=jax.ShapeDtypeStruct((BATCH, EMBED_DIM), jnp.float32),
    scratch_types=[
        pltpu.VMEM((B_PER_W,), jnp.int32),
        pltpu.SemaphoreType.DMA,
    ],
    compiler_params=pltpu.CompilerParams(use_tc_tiling_on_sc=True),
)
def _gather_kernel(idx_hbm, table_hbm, out_hbm, idx_v, sem):
    wid = lax.axis_index("s") * NUM_CORES + lax.axis_index("c")
    base = wid * B_PER_W
    pltpu.sync_copy(idx_hbm.at[pl.ds(base, B_PER_W)], idx_v)

    def fire(c, carry):
        vec = idx_v[pl.ds(c * LANES, LANES)]
        for j in range(LANES):
            i = vec[j]
            pltpu.make_async_copy(
                table_hbm.at[pl.ds(i, 1)],
                out_hbm.at[pl.ds(base + c * LANES + j, 1)],
                sem,
            ).start()
        return carry

    lax.fori_loop(0, B_PER_W // LANES, fire, 0)
    pltpu.make_async_copy(
        table_hbm.at[pl.ds(0, B_PER_W)],
        out_hbm.at[pl.ds(base, B_PER_W)],
        sem,
    ).wait()


def kernel(news_ids, embedding_table):
    idx = news_ids.astype(jnp.int32)
    return _gather_kernel(idx, embedding_table)


# speedup vs baseline: 6.2924x; 2.1920x over previous
import functools
import jax
import jax.numpy as jnp
from jax import lax
from jax.experimental import pallas as pl
from jax.experimental.pallas import tpu as pltpu
from jax.experimental.pallas import tpu_sc as plsc

VOCAB = 100000
EMBED_DIM = 300
BATCH = 4096

NUM_CORES = 2
NUM_SUBCORES = 16
LANES = 16
NUM_WORKERS = NUM_CORES * NUM_SUBCORES
B_PER_W = BATCH // NUM_WORKERS

_mesh = plsc.VectorSubcoreMesh(core_axis_name="c", subcore_axis_name="s")


@functools.partial(
    pl.kernel,
    mesh=_mesh,
    out_type=jax.ShapeDtypeStruct((BATCH, EMBED_DIM), jnp.float32),
    scratch_types=[
        pltpu.VMEM((B_PER_W,), jnp.int32),
        pltpu.VMEM((B_PER_W, EMBED_DIM), jnp.float32),
        pltpu.SemaphoreType.DMA,
    ],
    compiler_params=pltpu.CompilerParams(use_tc_tiling_on_sc=True),
)
def _gather_kernel(idx_hbm, table_hbm, out_hbm, idx_v, rows_v, sem):
    wid = lax.axis_index("s") * NUM_CORES + lax.axis_index("c")
    base = wid * B_PER_W
    pltpu.sync_copy(idx_hbm.at[pl.ds(base, B_PER_W)], idx_v)

    def fire(c, carry):
        vec = idx_v[pl.ds(c * LANES, LANES)]
        for j in range(LANES):
            i = vec[j]
            pltpu.make_async_copy(
                table_hbm.at[pl.ds(i, 1)],
                rows_v.at[pl.ds(c * LANES + j, 1)],
                sem,
            ).start()
        return carry

    lax.fori_loop(0, B_PER_W // LANES, fire, 0)
    pltpu.make_async_copy(
        table_hbm.at[pl.ds(0, B_PER_W)],
        rows_v,
        sem,
    ).wait()
    pltpu.sync_copy(rows_v, out_hbm.at[pl.ds(base, B_PER_W)])


def kernel(news_ids, embedding_table):
    idx = news_ids.astype(jnp.int32)
    return _gather_kernel(idx, embedding_table)


# speedup vs baseline: 11.2140x; 1.7822x over previous
import functools
import jax
import jax.numpy as jnp
from jax import lax
from jax.experimental import pallas as pl
from jax.experimental.pallas import tpu as pltpu
from jax.experimental.pallas import tpu_sc as plsc

VOCAB = 100000
EMBED_DIM = 300
BATCH = 4096

NUM_CORES = 2
NUM_SUBCORES = 16
LANES = 16
NUM_WORKERS = NUM_CORES * NUM_SUBCORES
ROWS_PER_W = 10

_mesh = plsc.VectorSubcoreMesh(core_axis_name="c", subcore_axis_name="s")


@functools.partial(
    pl.kernel,
    mesh=_mesh,
    out_type=jax.ShapeDtypeStruct((EMBED_DIM, BATCH), jnp.float32),
    scratch_types=[
        pltpu.VMEM((BATCH,), jnp.int32),
        pltpu.VMEM((VOCAB,), jnp.float32),
        pltpu.VMEM((BATCH,), jnp.float32),
        pltpu.SemaphoreType.DMA,
    ],
    compiler_params=pltpu.CompilerParams(
        use_tc_tiling_on_sc=True, needs_layout_passes=False
    ),
)
def _gather_t_kernel(idx_hbm, tab_t_hbm, out_t_hbm, idx_v, row_v, orow_v, sem):
    wid = lax.axis_index("s") * NUM_CORES + lax.axis_index("c")
    pltpu.sync_copy(idx_hbm, idx_v)

    def do_row(t, carry):
        j = t * NUM_WORKERS + wid

        @pl.when(j < EMBED_DIM)
        def _():
            pltpu.sync_copy(tab_t_hbm.at[j], row_v)

            def gather16(c, carry2):
                ivec = idx_v[pl.ds(c * LANES, LANES)]
                orow_v[pl.ds(c * LANES, LANES)] = plsc.load_gather(row_v, [ivec])
                return carry2

            lax.fori_loop(0, BATCH // LANES, gather16, 0)
            pltpu.sync_copy(orow_v, out_t_hbm.at[j])

        return carry

    lax.fori_loop(0, ROWS_PER_W, do_row, 0)


def kernel(news_ids, embedding_table):
    idx = news_ids.astype(jnp.int32)
    out_t = _gather_t_kernel(idx, embedding_table.T)
    return out_t.T


# speedup vs baseline: 12.2630x; 1.0935x over previous
import functools
import jax
import jax.numpy as jnp
from jax import lax
from jax.experimental import pallas as pl
from jax.experimental.pallas import tpu as pltpu
from jax.experimental.pallas import tpu_sc as plsc

VOCAB = 100000
EMBED_DIM = 300
BATCH = 4096

LO_SIZE = 49920
HI_BASE = LO_SIZE
HI_SIZE = 49920
TAIL_BASE = HI_BASE + HI_SIZE
TAIL_SIZE = VOCAB - TAIL_BASE

NUM_CORES = 2
NUM_SUBCORES = 16
LANES = 16
NUM_WORKERS = NUM_CORES * NUM_SUBCORES
ROWS_PER_W = 10
N_CHUNKS = BATCH // LANES

_mesh = plsc.VectorSubcoreMesh(core_axis_name="c", subcore_axis_name="s")


@functools.partial(
    pl.kernel,
    mesh=_mesh,
    out_type=jax.ShapeDtypeStruct((EMBED_DIM, BATCH), jnp.float32),
    scratch_types=[
        pltpu.VMEM((BATCH,), jnp.int32),
        pltpu.VMEM((BATCH + LANES,), jnp.int32),
        pltpu.VMEM((BATCH + LANES,), jnp.int32),
        pltpu.VMEM((LO_SIZE,), jnp.float32),
        pltpu.VMEM((HI_SIZE,), jnp.float32),
        pltpu.VMEM((TAIL_SIZE,), jnp.float32),
        pltpu.VMEM((BATCH,), jnp.float32),
        pltpu.VMEM((BATCH,), jnp.float32),
        pltpu.SemaphoreType.DMA,
        pltpu.SemaphoreType.DMA,
        pltpu.SemaphoreType.DMA,
        pltpu.SemaphoreType.DMA,
        pltpu.SemaphoreType.DMA,
    ],
    compiler_params=pltpu.CompilerParams(
        use_tc_tiling_on_sc=True, needs_layout_passes=False
    ),
)
def _gather_t_kernel(idx_hbm, tab_t_hbm, tail_t_hbm, out_t_hbm,
                     idx_v, part_val, part_pos, row_a, row_b, tail_v,
                     orow_a, orow_b, sem_a, sem_b, sem_t, sem_oa, sem_ob):
    wid = lax.axis_index("s") * NUM_CORES + lax.axis_index("c")
    pltpu.sync_copy(idx_hbm, idx_v)

    def piece(j, base, size, row_v, sem):
        return pltpu.make_async_copy(
            tab_t_hbm.at[j].at[pl.ds(base, size)], row_v, sem
        )

    def tail_piece(j):
        return pltpu.make_async_copy(tail_t_hbm.at[j], tail_v, sem_t)

    piece(wid, 0, LO_SIZE, row_a, sem_a).start()
    piece(wid, HI_BASE, HI_SIZE, row_b, sem_b).start()
    tail_piece(wid).start()

    iota = lax.iota(jnp.int32, LANES)

    def part_pass(lo_bound, hi_bound):
        def step(c, cnt):
            ivec = idx_v[pl.ds(c * LANES, LANES)]
            pvec = c * LANES + iota
            m = (ivec >= lo_bound) & (ivec < hi_bound)
            plsc.store_compressed(part_val.at[pl.ds(cnt, LANES)],
                                  ivec - lo_bound, mask=m)
            plsc.store_compressed(part_pos.at[pl.ds(cnt, LANES)], pvec, mask=m)
            return cnt + plsc.all_reduce_population_count(m)[0]
        return step

    n_lo = lax.fori_loop(0, N_CHUNKS, part_pass(0, HI_BASE), 0)
    n_lohi = lax.fori_loop(0, N_CHUNKS, part_pass(HI_BASE, TAIL_BASE), n_lo)
    lax.fori_loop(0, N_CHUNKS, part_pass(TAIL_BASE, VOCAB), n_lohi)

    def gather_span(row_v, orow_v, c_from, c_to, lane_from, lane_to):
        def chunk(c, carry):
            base = c * LANES
            vals = part_val[pl.ds(base, LANES)]
            pos = part_pos[pl.ds(base, LANES)]
            lane = base + iota
            m = (lane >= lane_from) & (lane < lane_to)
            g = plsc.load_gather(row_v, [vals], mask=m)
            plsc.store_scatter(orow_v, [pos], g, mask=m)
            return carry
        lax.fori_loop(c_from, c_to, chunk, 0)

    def out_wait(orow_v, sem):
        pltpu.make_async_copy(orow_v, out_t_hbm.at[0], sem).wait()

    def body(i, carry):
        for par in (0, 1):
            t = 2 * i + par
            j = t * NUM_WORKERS + wid
            jn = j + NUM_WORKERS
            orow_v = orow_a if par == 0 else orow_b
            sem_o = sem_oa if par == 0 else sem_ob

            @pl.when(i >= 1)
            def _():
                out_wait(orow_v, sem_o)

            @pl.when(j < EMBED_DIM)
            def _():
                piece(j, 0, LO_SIZE, row_a, sem_a).wait()
                gather_span(row_a, orow_v, 0, (n_lo + LANES - 1) // LANES,
                            0, n_lo)

            @pl.when(jn < EMBED_DIM)
            def _():
                piece(jn, 0, LO_SIZE, row_a, sem_a).start()

            @pl.when(j < EMBED_DIM)
            def _():
                tail_piece(j).wait()
                gather_span(tail_v, orow_v, n_lohi // LANES, N_CHUNKS,
                            n_lohi, BATCH)

            @pl.when(jn < EMBED_DIM)
            def _():
                tail_piece(jn).start()

            @pl.when(j < EMBED_DIM)
            def _():
                piece(j, HI_BASE, HI_SIZE, row_b, sem_b).wait()
                gather_span(row_b, orow_v, n_lo // LANES,
                            (n_lohi + LANES - 1) // LANES, n_lo, n_lohi)

            @pl.when(jn < EMBED_DIM)
            def _():
                piece(jn, HI_BASE, HI_SIZE, row_b, sem_b).start()

            @pl.when(j < EMBED_DIM)
            def _():
                pltpu.make_async_copy(orow_v, out_t_hbm.at[j], sem_o).start()

        return carry

    lax.fori_loop(0, ROWS_PER_W // 2, body, 0)

    out_wait(orow_a, sem_oa)

    @pl.when((ROWS_PER_W - 1) * NUM_WORKERS + wid < EMBED_DIM)
    def _():
        out_wait(orow_b, sem_ob)


def kernel(news_ids, embedding_table):
    idx = news_ids.astype(jnp.int32)
    tab_t = embedding_table.T
    tail_t = embedding_table[TAIL_BASE:, :].T
    out_t = _gather_t_kernel(idx, tab_t, tail_t)
    return out_t.T
